# baseline (speedup 1.0000x reference)
import functools

import jax
import jax.numpy as jnp
from jax import lax
from jax.experimental import pallas as pl
from jax.experimental.pallas import tpu as pltpu
from jax.experimental.pallas import tpu_sc as plsc

N_NODES = 10000
N_EDGES = 320000
D_IN = 128
H_DIM = 32
W_AUG = H_DIM + 8

NC = 2
NS = 16
NW = NC * NS

CHUNK = 128
TOT_CHUNKS = 2560
E_PAD = TOT_CHUNKS * CHUNK
CH0 = 160
CH1 = TOT_CHUNKS // NS - CH0
N_PAD = 10112
NBUF = 5
ROWS_PER_TILE = N_PAD // NS


def _tc1_body(x_ref, wl_ref, wr_ref, b_ref, y_ref, r_ref):
    x = x_ref[...]
    y = jnp.dot(x, wl_ref[...], preferred_element_type=jnp.float32)
    yaug = jnp.concatenate(
        [y, jnp.ones((N_PAD, W_AUG - H_DIM), jnp.float32)], axis=1
    )
    row = lax.broadcasted_iota(jnp.int32, (N_PAD, 1), 0)
    y_ref[...] = jnp.where(row < N_NODES, yaug, 0.0)
    r_ref[...] = (
        jnp.dot(x, wr_ref[...], preferred_element_type=jnp.float32) + b_ref[...]
    )


def _tc_stage1(x_pad, W1l, W1r, b1):
    return pl.pallas_call(
        _tc1_body,
        out_shape=(
            jax.ShapeDtypeStruct((N_PAD, W_AUG), jnp.float32),
            jax.ShapeDtypeStruct((N_PAD, H_DIM), jnp.float32),
        ),
    )(x_pad, W1l, W1r, b1)


def _sc_edge_body(width, y_hbm, src_hbm, dst_hbm, zeros_hbm, sums_hbm,
                  src_v, dst_v, rows_v, zbuf_v, acc_sh, *sems):
    gsems, ssems = sems[:NBUF], sems[NBUF:]
    c = lax.axis_index("c")
    s = lax.axis_index("s")

    pltpu.sync_copy(zeros_hbm, zbuf_v)
    pltpu.sync_copy(zbuf_v, acc_sh.at[pl.ds(s * ROWS_PER_TILE, ROWS_PER_TILE)])
    plsc.subcore_barrier()

    def run(base, nch):
        pltpu.sync_copy(src_hbm.at[pl.ds(base, nch)], src_v.at[pl.ds(0, nch)])
        pltpu.sync_copy(dst_hbm.at[pl.ds(base, nch)], dst_v.at[pl.ds(0, nch)])

        for b in range(NBUF):
            pltpu.async_copy(y_hbm.at[src_v.at[b]], rows_v.at[b], gsems[b])

        def group_step(g, carry):
            scatters = []
            for b in range(NBUF):
                j = g * NBUF + b
                pltpu.make_async_copy(
                    y_hbm.at[src_v.at[j]], rows_v.at[b], gsems[b]
                ).wait()
                scatters.append(
                    pltpu.async_copy(
                        rows_v.at[b], acc_sh.at[dst_v.at[j]], ssems[b],
                        add=True,
                    )
                )
            for b in range(NBUF):
                j = g * NBUF + b
                scatters[b].wait()

                @pl.when(g < nch // NBUF - 1)
                def _():
                    pltpu.async_copy(
                        y_hbm.at[src_v.at[j + NBUF]], rows_v.at[b], gsems[b]
                    )

            return carry

        lax.fori_loop(0, nch // NBUF, group_step, 0)

    @pl.when(c == 0)
    def _():
        run(s * CH0, CH0)

    if CH1 > 0:
        @pl.when(c == 1)
        def _():
            run(NS * CH0 + s * CH1, CH1)

    plsc.subcore_barrier()

    sl = pl.ds(s * ROWS_PER_TILE, ROWS_PER_TILE)
    pltpu.sync_copy(acc_sh.at[sl], zbuf_v)
    pltpu.sync_copy(zbuf_v, sums_hbm.at[c].at[sl])


def _sc_edge_pass(y, src2d, dst2d, width):
    mesh = plsc.VectorSubcoreMesh(
        core_axis_name="c", subcore_axis_name="s", num_cores=NC, num_subcores=NS
    )
    scratch = [
        pltpu.VMEM((max(CH0, CH1), CHUNK), jnp.int32),
        pltpu.VMEM((max(CH0, CH1), CHUNK), jnp.int32),
        pltpu.VMEM((NBUF, CHUNK, width), jnp.float32),
        pltpu.VMEM((ROWS_PER_TILE, width), jnp.float32),
        pltpu.VMEM_SHARED((N_PAD, width), jnp.float32),
    ] + [pltpu.SemaphoreType.DMA] * (2 * NBUF)
    zeros = jnp.zeros((ROWS_PER_TILE, width), jnp.float32)
    fn = pl.kernel(
        functools.partial(_sc_edge_body, width),
        out_type=jax.ShapeDtypeStruct((NC, N_PAD, width), jnp.float32),
        mesh=mesh,
        scratch_types=scratch,
        compiler_params=pltpu.CompilerParams(use_tc_tiling_on_sc=False),
    )
    return fn(y, src2d, dst2d, zeros)


def _tc2_body(s_ref, r_ref, wl_ref, wr_ref, b_ref, y_ref, r2_ref):
    ssum = s_ref[0, :, :H_DIM] + s_ref[1, :, :H_DIM]
    cnt = s_ref[0, :, H_DIM:H_DIM + 1] + s_ref[1, :, H_DIM:H_DIM + 1]
    mean = ssum / jnp.maximum(cnt, 1.0)
    h = jnp.maximum(mean + r_ref[...], 0.0)
    y2 = jnp.dot(h, wl_ref[...], preferred_element_type=jnp.float32)
    row = lax.broadcasted_iota(jnp.int32, (N_PAD, 1), 0)
    y_ref[...] = jnp.where(row < N_NODES, y2, 0.0)
    r2_ref[...] = (
        jnp.dot(h, wr_ref[...], preferred_element_type=jnp.float32) + b_ref[...]
    )


def _tc_stage2(sums1, r1, W2l, W2r, b2):
    return pl.pallas_call(
        _tc2_body,
        out_shape=(
            jax.ShapeDtypeStruct((N_PAD, H_DIM), jnp.float32),
            jax.ShapeDtypeStruct((N_PAD, H_DIM), jnp.float32),
        ),
    )(sums1, r1, W2l, W2r, b2)


def _tc3_body(s2_ref, s1_ref, r_ref, w_ref, b_ref, o_ref):
    ssum = s2_ref[0] + s2_ref[1]
    cnt = s1_ref[0, :, H_DIM:H_DIM + 1] + s1_ref[1, :, H_DIM:H_DIM + 1]
    mean = ssum / jnp.maximum(cnt, 1.0)
    h2 = mean + r_ref[...]
    o_ref[...] = (
        jnp.sum(h2 * w_ref[...], axis=1, keepdims=True) + b_ref[...]
    )


def _tc_stage3(sums2, sums1, r2, wvec, blin):
    return pl.pallas_call(
        _tc3_body,
        out_shape=jax.ShapeDtypeStruct((N_PAD, 1), jnp.float32),
    )(sums2, sums1, r2, wvec, blin)


def kernel(x, edge_index, W1l, b1, W1r, W2l, b2, W2r, Wlin, blin):
    x_pad = jnp.zeros((N_PAD, D_IN), x.dtype).at[:N_NODES].set(x)

    pad = E_PAD - N_EDGES
    src = jnp.concatenate(
        [edge_index[0], jnp.full((pad,), N_NODES, jnp.int32)]
    ).reshape(E_PAD // CHUNK, CHUNK)
    dst = jnp.concatenate(
        [edge_index[1], jnp.arange(pad, dtype=jnp.int32) % N_PAD]
    ).reshape(E_PAD // CHUNK, CHUNK)

    y1, r1 = _tc_stage1(x_pad, W1l, W1r, b1.reshape(1, H_DIM))
    sums1 = _sc_edge_pass(y1, src, dst, W_AUG)
    y2, r2 = _tc_stage2(sums1, r1, W2l, W2r, b2.reshape(1, H_DIM))
    sums2 = _sc_edge_pass(y2, src, dst, H_DIM)
    out = _tc_stage3(sums2, sums1, r2, Wlin.reshape(1, H_DIM),
                     blin.reshape(1, 1))
    return out[:N_NODES]


# speedup vs baseline: 1.2267x; 1.2267x over previous
import functools

import jax
import jax.numpy as jnp
from jax import lax
from jax.experimental import pallas as pl
from jax.experimental.pallas import tpu as pltpu
from jax.experimental.pallas import tpu_sc as plsc

N_NODES = 10000
N_EDGES = 320000
D_IN = 128
H_DIM = 32
W_AUG = H_DIM + 8

NC = 2
NS = 16
NW = NC * NS

CHUNK = 128
TOT_CHUNKS = 2560
E_PAD = TOT_CHUNKS * CHUNK
CH0 = 112
CH1 = TOT_CHUNKS // NS - CH0
N_PAD = 10112
NBUF = 8
ROWS_PER_TILE = N_PAD // NS


def _tc1_body(x_ref, wl_ref, wr_ref, b_ref, y_ref, r_ref):
    x = x_ref[...]
    y = jnp.dot(x, wl_ref[...], preferred_element_type=jnp.float32)
    yaug = jnp.concatenate(
        [y, jnp.ones((N_PAD, W_AUG - H_DIM), jnp.float32)], axis=1
    )
    row = lax.broadcasted_iota(jnp.int32, (N_PAD, 1), 0)
    y_ref[...] = jnp.where(row < N_NODES, yaug, 0.0)
    r_ref[...] = (
        jnp.dot(x, wr_ref[...], preferred_element_type=jnp.float32) + b_ref[...]
    )


def _tc_stage1(x_pad, W1l, W1r, b1):
    return pl.pallas_call(
        _tc1_body,
        out_shape=(
            jax.ShapeDtypeStruct((N_PAD, W_AUG), jnp.float32),
            jax.ShapeDtypeStruct((N_PAD, H_DIM), jnp.float32),
        ),
    )(x_pad, W1l, W1r, b1)


def _sc_edge_body(width, y_hbm, src_hbm, dst_hbm, zeros_hbm, sums_hbm,
                  src_v, dst_v, rows_v, zbuf_v, acc_sh, *sems):
    gsems, ssems = sems[:NBUF], sems[NBUF:]
    c = lax.axis_index("c")
    s = lax.axis_index("s")

    pltpu.sync_copy(zeros_hbm, zbuf_v)
    pltpu.sync_copy(zbuf_v, acc_sh.at[pl.ds(s * ROWS_PER_TILE, ROWS_PER_TILE)])
    plsc.subcore_barrier()

    def run(base, nch):
        pltpu.sync_copy(src_hbm.at[pl.ds(base, nch)], src_v.at[pl.ds(0, nch)])
        pltpu.sync_copy(dst_hbm.at[pl.ds(base, nch)], dst_v.at[pl.ds(0, nch)])

        for b in range(NBUF):
            pltpu.async_copy(y_hbm.at[src_v.at[b]], rows_v.at[b], gsems[b])

        def group_step(g, carry):
            scatters = []
            for b in range(NBUF):
                j = g * NBUF + b
                pltpu.make_async_copy(
                    y_hbm.at[src_v.at[j]], rows_v.at[b], gsems[b]
                ).wait()
                scatters.append(
                    pltpu.async_copy(
                        rows_v.at[b], acc_sh.at[dst_v.at[j]], ssems[b],
                        add=True,
                    )
                )
            for b in range(NBUF):
                j = g * NBUF + b
                scatters[b].wait()

                @pl.when(g < nch // NBUF - 1)
                def _():
                    pltpu.async_copy(
                        y_hbm.at[src_v.at[j + NBUF]], rows_v.at[b], gsems[b]
                    )

            return carry

        lax.fori_loop(0, nch // NBUF, group_step, 0)

    @pl.when(c == 0)
    def _():
        run(s * CH0, CH0)

    if CH1 > 0:
        @pl.when(c == 1)
        def _():
            run(NS * CH0 + s * CH1, CH1)

    plsc.subcore_barrier()

    sl = pl.ds(s * ROWS_PER_TILE, ROWS_PER_TILE)
    pltpu.sync_copy(acc_sh.at[sl], zbuf_v)
    pltpu.sync_copy(zbuf_v, sums_hbm.at[c].at[sl])


def _sc_edge_pass(y, src2d, dst2d, width):
    mesh = plsc.VectorSubcoreMesh(
        core_axis_name="c", subcore_axis_name="s", num_cores=NC, num_subcores=NS
    )
    scratch = [
        pltpu.VMEM((max(CH0, CH1), CHUNK), jnp.int32),
        pltpu.VMEM((max(CH0, CH1), CHUNK), jnp.int32),
        pltpu.VMEM((NBUF, CHUNK, width), jnp.float32),
        pltpu.VMEM((ROWS_PER_TILE, width), jnp.float32),
        pltpu.VMEM_SHARED((N_PAD, width), jnp.float32),
    ] + [pltpu.SemaphoreType.DMA] * (2 * NBUF)
    zeros = jnp.zeros((ROWS_PER_TILE, width), jnp.float32)
    fn = pl.kernel(
        functools.partial(_sc_edge_body, width),
        out_type=jax.ShapeDtypeStruct((NC, N_PAD, width), jnp.float32),
        mesh=mesh,
        scratch_types=scratch,
        compiler_params=pltpu.CompilerParams(use_tc_tiling_on_sc=False),
    )
    return fn(y, src2d, dst2d, zeros)


def _tc2_body(s_ref, r_ref, wl_ref, wr_ref, b_ref, y_ref, r2_ref):
    ssum = s_ref[0, :, :H_DIM] + s_ref[1, :, :H_DIM]
    cnt = s_ref[0, :, H_DIM:H_DIM + 1] + s_ref[1, :, H_DIM:H_DIM + 1]
    mean = ssum / jnp.maximum(cnt, 1.0)
    h = jnp.maximum(mean + r_ref[...], 0.0)
    y2 = jnp.dot(h, wl_ref[...], preferred_element_type=jnp.float32)
    row = lax.broadcasted_iota(jnp.int32, (N_PAD, 1), 0)
    y_ref[...] = jnp.where(row < N_NODES, y2, 0.0)
    r2_ref[...] = (
        jnp.dot(h, wr_ref[...], preferred_element_type=jnp.float32) + b_ref[...]
    )


def _tc_stage2(sums1, r1, W2l, W2r, b2):
    return pl.pallas_call(
        _tc2_body,
        out_shape=(
            jax.ShapeDtypeStruct((N_PAD, H_DIM), jnp.float32),
            jax.ShapeDtypeStruct((N_PAD, H_DIM), jnp.float32),
        ),
    )(sums1, r1, W2l, W2r, b2)


def _tc3_body(s2_ref, s1_ref, r_ref, w_ref, b_ref, o_ref):
    ssum = s2_ref[0] + s2_ref[1]
    cnt = s1_ref[0, :, H_DIM:H_DIM + 1] + s1_ref[1, :, H_DIM:H_DIM + 1]
    mean = ssum / jnp.maximum(cnt, 1.0)
    h2 = mean + r_ref[...]
    o_ref[...] = (
        jnp.sum(h2 * w_ref[...], axis=1, keepdims=True) + b_ref[...]
    )


def _tc_stage3(sums2, sums1, r2, wvec, blin):
    return pl.pallas_call(
        _tc3_body,
        out_shape=jax.ShapeDtypeStruct((N_PAD, 1), jnp.float32),
    )(sums2, sums1, r2, wvec, blin)


def kernel(x, edge_index, W1l, b1, W1r, W2l, b2, W2r, Wlin, blin):
    x_pad = jnp.zeros((N_PAD, D_IN), x.dtype).at[:N_NODES].set(x)

    pad = E_PAD - N_EDGES
    src = jnp.concatenate(
        [edge_index[0], jnp.full((pad,), N_NODES, jnp.int32)]
    ).reshape(E_PAD // CHUNK, CHUNK)
    dst = jnp.concatenate(
        [edge_index[1], jnp.arange(pad, dtype=jnp.int32) % N_PAD]
    ).reshape(E_PAD // CHUNK, CHUNK)

    y1, r1 = _tc_stage1(x_pad, W1l, W1r, b1.reshape(1, H_DIM))
    sums1 = _sc_edge_pass(y1, src, dst, W_AUG)
    y2, r2 = _tc_stage2(sums1, r1, W2l, W2r, b2.reshape(1, H_DIM))
    sums2 = _sc_edge_pass(y2, src, dst, H_DIM)
    out = _tc_stage3(sums2, sums1, r2, Wlin.reshape(1, H_DIM),
                     blin.reshape(1, 1))
    return out[:N_NODES]


# speedup vs baseline: 1.2382x; 1.0094x over previous
import functools

import jax
import jax.numpy as jnp
from jax import lax
from jax.experimental import pallas as pl
from jax.experimental.pallas import tpu as pltpu
from jax.experimental.pallas import tpu_sc as plsc

N_NODES = 10000
N_EDGES = 320000
D_IN = 128
H_DIM = 32
W_AUG = H_DIM + 8

NC = 2
NS = 16
NW = NC * NS

CHUNK = 128
TOT_CHUNKS = 2560
E_PAD = TOT_CHUNKS * CHUNK
CH0 = 120
CH1 = TOT_CHUNKS // NS - CH0
N_PAD = 10112
NBUF = 8
ROWS_PER_TILE = N_PAD // NS


def _tc1_body(x_ref, wl_ref, wr_ref, b_ref, y_ref, r_ref):
    x = x_ref[...]
    y = jnp.dot(x, wl_ref[...], preferred_element_type=jnp.float32)
    yaug = jnp.concatenate(
        [y, jnp.ones((N_PAD, W_AUG - H_DIM), jnp.float32)], axis=1
    )
    row = lax.broadcasted_iota(jnp.int32, (N_PAD, 1), 0)
    y_ref[...] = jnp.where(row < N_NODES, yaug, 0.0)
    r_ref[...] = (
        jnp.dot(x, wr_ref[...], preferred_element_type=jnp.float32) + b_ref[...]
    )


def _tc_stage1(x_pad, W1l, W1r, b1):
    return pl.pallas_call(
        _tc1_body,
        out_shape=(
            jax.ShapeDtypeStruct((N_PAD, W_AUG), jnp.float32),
            jax.ShapeDtypeStruct((N_PAD, H_DIM), jnp.float32),
        ),
    )(x_pad, W1l, W1r, b1)


def _sc_edge_body(width, y_hbm, src_hbm, dst_hbm, zeros_hbm, sums_hbm,
                  src_v, dst_v, rows_v, acc_sh, *sems):
    gsems, ssems = sems[:NBUF], sems[NBUF:]
    c = lax.axis_index("c")
    s = lax.axis_index("s")

    pltpu.sync_copy(zeros_hbm, acc_sh.at[pl.ds(s * ROWS_PER_TILE, ROWS_PER_TILE)])
    plsc.subcore_barrier()

    def run(base, nch):
        pltpu.sync_copy(src_hbm.at[pl.ds(base, nch)], src_v.at[pl.ds(0, nch)])
        pltpu.sync_copy(dst_hbm.at[pl.ds(base, nch)], dst_v.at[pl.ds(0, nch)])

        for b in range(NBUF):
            pltpu.async_copy(y_hbm.at[src_v.at[b]], rows_v.at[b], gsems[b])

        def group_step(g, carry):
            scatters = []
            for b in range(NBUF):
                j = g * NBUF + b
                pltpu.make_async_copy(
                    y_hbm.at[src_v.at[j]], rows_v.at[b], gsems[b]
                ).wait()
                scatters.append(
                    pltpu.async_copy(
                        rows_v.at[b], acc_sh.at[dst_v.at[j]], ssems[b],
                        add=True,
                    )
                )
            for b in range(NBUF):
                j = g * NBUF + b
                scatters[b].wait()

                @pl.when(g < nch // NBUF - 1)
                def _():
                    pltpu.async_copy(
                        y_hbm.at[src_v.at[j + NBUF]], rows_v.at[b], gsems[b]
                    )

            return carry

        lax.fori_loop(0, nch // NBUF, group_step, 0)

    @pl.when(c == 0)
    def _():
        run(s * CH0, CH0)

    if CH1 > 0:
        @pl.when(c == 1)
        def _():
            run(NS * CH0 + s * CH1, CH1)

    plsc.subcore_barrier()

    sl = pl.ds(s * ROWS_PER_TILE, ROWS_PER_TILE)
    pltpu.sync_copy(acc_sh.at[sl], sums_hbm.at[c].at[sl])


def _sc_edge_pass(y, src2d, dst2d, width):
    mesh = plsc.VectorSubcoreMesh(
        core_axis_name="c", subcore_axis_name="s", num_cores=NC, num_subcores=NS
    )
    scratch = [
        pltpu.VMEM((max(CH0, CH1), CHUNK), jnp.int32),
        pltpu.VMEM((max(CH0, CH1), CHUNK), jnp.int32),
        pltpu.VMEM((NBUF, CHUNK, width), jnp.float32),
        pltpu.VMEM_SHARED((N_PAD, width), jnp.float32),
    ] + [pltpu.SemaphoreType.DMA] * (2 * NBUF)
    zeros = jnp.zeros((ROWS_PER_TILE, width), jnp.float32)
    fn = pl.kernel(
        functools.partial(_sc_edge_body, width),
        out_type=jax.ShapeDtypeStruct((NC, N_PAD, width), jnp.float32),
        mesh=mesh,
        scratch_types=scratch,
        compiler_params=pltpu.CompilerParams(use_tc_tiling_on_sc=False),
    )
    return fn(y, src2d, dst2d, zeros)


def _tc2_body(s_ref, r_ref, wl_ref, wr_ref, b_ref, y_ref, r2_ref):
    ssum = s_ref[0, :, :H_DIM] + s_ref[1, :, :H_DIM]
    cnt = s_ref[0, :, H_DIM:H_DIM + 1] + s_ref[1, :, H_DIM:H_DIM + 1]
    mean = ssum / jnp.maximum(cnt, 1.0)
    h = jnp.maximum(mean + r_ref[...], 0.0)
    y2 = jnp.dot(h, wl_ref[...], preferred_element_type=jnp.float32)
    row = lax.broadcasted_iota(jnp.int32, (N_PAD, 1), 0)
    y_ref[...] = jnp.where(row < N_NODES, y2, 0.0)
    r2_ref[...] = (
        jnp.dot(h, wr_ref[...], preferred_element_type=jnp.float32) + b_ref[...]
    )


def _tc_stage2(sums1, r1, W2l, W2r, b2):
    return pl.pallas_call(
        _tc2_body,
        out_shape=(
            jax.ShapeDtypeStruct((N_PAD, H_DIM), jnp.float32),
            jax.ShapeDtypeStruct((N_PAD, H_DIM), jnp.float32),
        ),
    )(sums1, r1, W2l, W2r, b2)


def _tc3_body(s2_ref, s1_ref, r_ref, w_ref, b_ref, o_ref):
    ssum = s2_ref[0] + s2_ref[1]
    cnt = s1_ref[0, :, H_DIM:H_DIM + 1] + s1_ref[1, :, H_DIM:H_DIM + 1]
    mean = ssum / jnp.maximum(cnt, 1.0)
    h2 = mean + r_ref[...]
    o_ref[...] = (
        jnp.sum(h2 * w_ref[...], axis=1, keepdims=True) + b_ref[...]
    )


def _tc_stage3(sums2, sums1, r2, wvec, blin):
    return pl.pallas_call(
        _tc3_body,
        out_shape=jax.ShapeDtypeStruct((N_PAD, 1), jnp.float32),
    )(sums2, sums1, r2, wvec, blin)


def kernel(x, edge_index, W1l, b1, W1r, W2l, b2, W2r, Wlin, blin):
    x_pad = jnp.zeros((N_PAD, D_IN), x.dtype).at[:N_NODES].set(x)

    pad = E_PAD - N_EDGES
    src = jnp.concatenate(
        [edge_index[0], jnp.full((pad,), N_NODES, jnp.int32)]
    ).reshape(E_PAD // CHUNK, CHUNK)
    dst = jnp.concatenate(
        [edge_index[1], jnp.arange(pad, dtype=jnp.int32) % N_PAD]
    ).reshape(E_PAD // CHUNK, CHUNK)

    y1, r1 = _tc_stage1(x_pad, W1l, W1r, b1.reshape(1, H_DIM))
    sums1 = _sc_edge_pass(y1, src, dst, W_AUG)
    y2, r2 = _tc_stage2(sums1, r1, W2l, W2r, b2.reshape(1, H_DIM))
    sums2 = _sc_edge_pass(y2, src, dst, H_DIM)
    out = _tc_stage3(sums2, sums1, r2, Wlin.reshape(1, H_DIM),
                     blin.reshape(1, 1))
    return out[:N_NODES]


# speedup vs baseline: 1.2545x; 1.0131x over previous
import functools

import jax
import jax.numpy as jnp
from jax import lax
from jax.experimental import pallas as pl
from jax.experimental.pallas import tpu as pltpu
from jax.experimental.pallas import tpu_sc as plsc

N_NODES = 10000
N_EDGES = 320000
D_IN = 128
H_DIM = 32
W_AUG = H_DIM + 8

NC = 2
NS = 16
NW = NC * NS

CHUNK = 128
TOT_CHUNKS = 2560
E_PAD = TOT_CHUNKS * CHUNK
CH0 = 120
CH1 = TOT_CHUNKS // NS - CH0
N_PAD = 10112
NBUF = 8
ROWS_PER_TILE = N_PAD // NS


def _tc1_body(x_ref, wl_ref, wr_ref, b_ref, y_ref, r_ref):
    x = x_ref[...]
    y = jnp.dot(x, wl_ref[...], preferred_element_type=jnp.float32)
    yaug = jnp.concatenate(
        [y, jnp.ones((N_PAD, W_AUG - H_DIM), jnp.float32)], axis=1
    )
    row = lax.broadcasted_iota(jnp.int32, (N_PAD, 1), 0)
    y_ref[...] = jnp.where(row < N_NODES, yaug, 0.0)
    r_ref[...] = (
        jnp.dot(x, wr_ref[...], preferred_element_type=jnp.float32) + b_ref[...]
    )


def _tc_stage1(x_pad, W1l, W1r, b1):
    return pl.pallas_call(
        _tc1_body,
        out_shape=(
            jax.ShapeDtypeStruct((N_PAD, W_AUG), jnp.float32),
            jax.ShapeDtypeStruct((N_PAD, H_DIM), jnp.float32),
        ),
    )(x_pad, W1l, W1r, b1)


def _sc_edge_body(width, y_hbm, src_hbm, dst_hbm, zeros_hbm, sums_hbm,
                  src_v, dst_v, rows_v, acc_sh, *sems):
    gsems, ssems = sems[:NBUF], sems[NBUF:]
    c = lax.axis_index("c")
    s = lax.axis_index("s")

    def run(base, nch):
        zero = pltpu.async_copy(
            zeros_hbm,
            acc_sh.at[pl.ds(s * ROWS_PER_TILE, ROWS_PER_TILE)],
            ssems[0],
        )
        pltpu.sync_copy(src_hbm.at[pl.ds(base, nch)], src_v.at[pl.ds(0, nch)])
        pltpu.sync_copy(dst_hbm.at[pl.ds(base, nch)], dst_v.at[pl.ds(0, nch)])

        for b in range(NBUF):
            pltpu.async_copy(y_hbm.at[src_v.at[b]], rows_v.at[b], gsems[b])
        zero.wait()
        plsc.subcore_barrier()

        def group_step(g, carry):
            scatters = []
            for b in range(NBUF):
                j = g * NBUF + b
                pltpu.make_async_copy(
                    y_hbm.at[src_v.at[j]], rows_v.at[b], gsems[b]
                ).wait()
                scatters.append(
                    pltpu.async_copy(
                        rows_v.at[b], acc_sh.at[dst_v.at[j]], ssems[b],
                        add=True,
                    )
                )
            for b in range(NBUF):
                j = g * NBUF + b
                scatters[b].wait()

                @pl.when(g < nch // NBUF - 1)
                def _():
                    pltpu.async_copy(
                        y_hbm.at[src_v.at[j + NBUF]], rows_v.at[b], gsems[b]
                    )

            return carry

        lax.fori_loop(0, nch // NBUF, group_step, 0)

    @pl.when(c == 0)
    def _():
        run(s * CH0, CH0)

    @pl.when(c == 1)
    def _():
        run(NS * CH0 + s * CH1, CH1)

    plsc.subcore_barrier()

    sl = pl.ds(s * ROWS_PER_TILE, ROWS_PER_TILE)
    pltpu.sync_copy(acc_sh.at[sl], sums_hbm.at[c].at[sl])


def _sc_edge_pass(y, src2d, dst2d, width):
    mesh = plsc.VectorSubcoreMesh(
        core_axis_name="c", subcore_axis_name="s", num_cores=NC, num_subcores=NS
    )
    scratch = [
        pltpu.VMEM((max(CH0, CH1), CHUNK), jnp.int32),
        pltpu.VMEM((max(CH0, CH1), CHUNK), jnp.int32),
        pltpu.VMEM((NBUF, CHUNK, width), jnp.float32),
        pltpu.VMEM_SHARED((N_PAD, width), jnp.float32),
    ] + [pltpu.SemaphoreType.DMA] * (2 * NBUF)
    zeros = jnp.zeros((ROWS_PER_TILE, width), jnp.float32)
    fn = pl.kernel(
        functools.partial(_sc_edge_body, width),
        out_type=jax.ShapeDtypeStruct((NC, N_PAD, width), jnp.float32),
        mesh=mesh,
        scratch_types=scratch,
        compiler_params=pltpu.CompilerParams(use_tc_tiling_on_sc=False),
    )
    return fn(y, src2d, dst2d, zeros)


def _tc2_body(s_ref, r_ref, wl_ref, wr_ref, b_ref, y_ref, r2_ref):
    ssum = s_ref[0, :, :H_DIM] + s_ref[1, :, :H_DIM]
    cnt = s_ref[0, :, H_DIM:H_DIM + 1] + s_ref[1, :, H_DIM:H_DIM + 1]
    mean = ssum / jnp.maximum(cnt, 1.0)
    h = jnp.maximum(mean + r_ref[...], 0.0)
    y2 = jnp.dot(h, wl_ref[...], preferred_element_type=jnp.float32)
    row = lax.broadcasted_iota(jnp.int32, (N_PAD, 1), 0)
    y_ref[...] = jnp.where(row < N_NODES, y2, 0.0)
    r2_ref[...] = (
        jnp.dot(h, wr_ref[...], preferred_element_type=jnp.float32) + b_ref[...]
    )


def _tc_stage2(sums1, r1, W2l, W2r, b2):
    return pl.pallas_call(
        _tc2_body,
        out_shape=(
            jax.ShapeDtypeStruct((N_PAD, H_DIM), jnp.float32),
            jax.ShapeDtypeStruct((N_PAD, H_DIM), jnp.float32),
        ),
    )(sums1, r1, W2l, W2r, b2)


def _tc3_body(s2_ref, s1_ref, r_ref, w_ref, b_ref, o_ref):
    ssum = s2_ref[0] + s2_ref[1]
    cnt = s1_ref[0, :, H_DIM:H_DIM + 1] + s1_ref[1, :, H_DIM:H_DIM + 1]
    mean = ssum / jnp.maximum(cnt, 1.0)
    h2 = mean + r_ref[...]
    o_ref[...] = (
        jnp.sum(h2 * w_ref[...], axis=1, keepdims=True) + b_ref[...]
    )


def _tc_stage3(sums2, sums1, r2, wvec, blin):
    return pl.pallas_call(
        _tc3_body,
        out_shape=jax.ShapeDtypeStruct((N_PAD, 1), jnp.float32),
    )(sums2, sums1, r2, wvec, blin)


def kernel(x, edge_index, W1l, b1, W1r, W2l, b2, W2r, Wlin, blin):
    x_pad = jnp.zeros((N_PAD, D_IN), x.dtype).at[:N_NODES].set(x)

    pad = E_PAD - N_EDGES
    src = jnp.concatenate(
        [edge_index[0], jnp.full((pad,), N_NODES, jnp.int32)]
    ).reshape(E_PAD // CHUNK, CHUNK)
    dst = jnp.concatenate(
        [edge_index[1], jnp.arange(pad, dtype=jnp.int32) % N_PAD]
    ).reshape(E_PAD // CHUNK, CHUNK)

    y1, r1 = _tc_stage1(x_pad, W1l, W1r, b1.reshape(1, H_DIM))
    sums1 = _sc_edge_pass(y1, src, dst, W_AUG)
    y2, r2 = _tc_stage2(sums1, r1, W2l, W2r, b2.reshape(1, H_DIM))
    sums2 = _sc_edge_pass(y2, src, dst, H_DIM)
    out = _tc_stage3(sums2, sums1, r2, Wlin.reshape(1, H_DIM),
                     blin.reshape(1, 1))
    return out[:N_NODES]
